# baseline (speedup 1.0000x reference)
import functools

import jax
import jax.numpy as jnp
from jax import lax
from jax.experimental import pallas as pl
from jax.experimental.pallas import tpu as pltpu
from jax.experimental.pallas import tpu_sc as plsc

N = 10000
E = 320000
D_LAB = 128
D_EMB = 768
D_ORG = 128
N_LABS = 1024
N_ORG = 2048
N_REL = 100

NC = 2
NS = 16
NW = NC * NS

N_PAD = 10240
ROWS_PER_W = N_PAD // NW
GCHUNK = 80
NGCH = ROWS_PER_W // GCHUNK
ECHUNK = 80
NCH = (E // NW) // ECHUNK
N_REL_PAD = 128

HP = 4
HORG = N_ORG // HP
CSTRIDE = N_REL
HWORDS = HORG * CSTRIDE


def _dense_pre_body(lab_ref, w1t_ref, b_ref, cemb_ref, w2t_ref, remb_ref,
                    dwt_ref, worgt_ref, dst_ref, rel_ref,
                    blab_ref, plab_ref, z_ref, key_ref):
    blab_ref[...] = jnp.dot(lab_ref[...], w1t_ref[...],
                            preferred_element_type=jnp.float32) + b_ref[...]
    plab_ref[...] = jnp.dot(cemb_ref[...], w2t_ref[...],
                            preferred_element_type=jnp.float32)
    prel = jnp.dot(remb_ref[...], dwt_ref[...],
                   preferred_element_type=jnp.float32, precision=lax.Precision.HIGHEST)
    z_ref[...] = jnp.dot(prel, worgt_ref[...],
                         preferred_element_type=jnp.float32, precision=lax.Precision.HIGHEST)
    key_ref[...] = dst_ref[...] * CSTRIDE + rel_ref[...]


_dense_pre = pl.pallas_call(
    _dense_pre_body,
    out_shape=(
        jax.ShapeDtypeStruct((N_PAD, D_LAB), jnp.float32),
        jax.ShapeDtypeStruct((N_LABS, D_LAB), jnp.float32),
        jax.ShapeDtypeStruct((N_REL_PAD, D_LAB), jnp.float32),
        jax.ShapeDtypeStruct((E // D_LAB, D_LAB), jnp.int32),
    ),
)


def _h_build_body(plab_hbm, blab_hbm, idx3d_hbm, h_hbm,
                  idx_v, g_v, b_v, semg, semb):
    cid = lax.axis_index("c")
    sid = lax.axis_index("s")
    wid = sid * NC + cid
    base = wid * ROWS_PER_W
    pltpu.sync_copy(idx3d_hbm.at[wid], idx_v)
    for ci in range(NGCH):
        pltpu.async_copy(plab_hbm.at[idx_v.at[ci]],
                         g_v.at[pl.ds(ci * GCHUNK, GCHUNK)], semg)
    pltpu.async_copy(blab_hbm.at[pl.ds(base, ROWS_PER_W)], b_v, semb)
    for ci in range(NGCH):
        pltpu.make_async_copy(plab_hbm.at[idx_v.at[ci]],
                              g_v.at[pl.ds(ci * GCHUNK, GCHUNK)], semg).wait()
    pltpu.make_async_copy(blab_hbm.at[pl.ds(base, ROWS_PER_W)], b_v,
                          semb).wait()

    def row_body(i, _):
        for j in range(D_LAB // 16):
            s = pl.ds(j * 16, 16)
            g_v[i, s] = jnp.maximum(g_v[i, s] + b_v[i, s], 0.0)
        return 0

    lax.fori_loop(0, ROWS_PER_W, row_body, 0)
    pltpu.sync_copy(g_v, h_hbm.at[pl.ds(base, ROWS_PER_W)])


_h_build = pl.kernel(
    _h_build_body,
    out_type=jax.ShapeDtypeStruct((N_PAD, D_LAB), jnp.float32),
    mesh=plsc.VectorSubcoreMesh(core_axis_name="c", subcore_axis_name="s",
                                num_cores=NC, num_subcores=NS),
    scratch_types=[
        pltpu.VMEM((NGCH, GCHUNK), jnp.int32),
        pltpu.VMEM((ROWS_PER_W, D_LAB), jnp.float32),
        pltpu.VMEM((ROWS_PER_W, D_LAB), jnp.float32),
        pltpu.SemaphoreType.DMA,
        pltpu.SemaphoreType.DMA,
    ],
)


def _edge_agg_body(h_hbm, src2d_hbm, dst2d_hbm, key_hbm, zeros_hbm, z2d_hbm,
                   out_hbm, outc_hbm, srcall_v, dstall_v, rowsa_v, hist_v,
                   agg_sh, semg0, semg1, sems0, sems1):
    sems_g = (semg0, semg1)
    sems_s = (sems0, sems1)
    cid = lax.axis_index("c")
    sid = lax.axis_index("s")
    wid = sid * NC + cid

    base_r = sid * (N_ORG // NS)
    pltpu.sync_copy(z2d_hbm, agg_sh.at[pl.ds(base_r, N_ORG // NS)])

    pltpu.sync_copy(src2d_hbm.at[wid], srcall_v)
    pltpu.sync_copy(dst2d_hbm.at[wid], dstall_v)
    plsc.subcore_barrier()

    nch = NCH

    def fire_gather(t, slot):
        pltpu.async_copy(h_hbm.at[srcall_v.at[t]], rowsa_v.at[slot],
                         sems_g[slot])

    def wait_gather(t, slot):
        pltpu.make_async_copy(h_hbm.at[srcall_v.at[t]], rowsa_v.at[slot],
                              sems_g[slot]).wait()

    def fire_scatter(t, slot):
        pltpu.async_copy(rowsa_v.at[slot], agg_sh.at[dstall_v.at[t]],
                         sems_s[slot], add=True)

    def wait_scatter(t, slot):
        pltpu.make_async_copy(rowsa_v.at[slot], agg_sh.at[dstall_v.at[t]],
                              sems_s[slot]).wait()

    fire_gather(0, 0)

    def pair(p, _):
        for half in range(2):
            t = 2 * p + half
            nslot = (half + 1) % 2

            @pl.when((t + 1 < nch) & (t >= 1))
            def _():
                wait_scatter(t - 1, nslot)

            @pl.when(t + 1 < nch)
            def _():
                fire_gather(t + 1, nslot)

            @pl.when(t < nch)
            def _():
                wait_gather(t, half)
                fire_scatter(t, half)
        return 0

    lax.fori_loop(0, (nch + 1) // 2, pair, 0)
    wait_scatter(nch - 2, (nch - 2) % 2)
    wait_scatter(nch - 1, (nch - 1) % 2)

    keyall_v = srcall_v
    pltpu.sync_copy(key_hbm.at[wid], keyall_v)

    ones16 = jnp.ones((16,), jnp.float32)

    for p in range(HP):
        pltpu.sync_copy(zeros_hbm, hist_v)
        klo = p * HORG * CSTRIDE

        def hbody(i, _):
            for g in range(ECHUNK // 16):
                k = keyall_v[i, pl.ds(g * 16, 16)]
                idx = k - klo
                m = (k >= klo) & (k < klo + HWORDS)
                plsc.addupdate_scatter(hist_v, [idx], ones16, mask=m)
            return 0

        lax.fori_loop(0, NCH, hbody, 0)
        pltpu.sync_copy(
            hist_v, outc_hbm.at[pl.ds(wid * HP * HWORDS + p * HWORDS, HWORDS)])

    plsc.subcore_barrier()

    rpt = N_ORG // NS
    pltpu.sync_copy(agg_sh.at[pl.ds(sid * rpt, rpt)],
                    out_hbm.at[pl.ds(cid * N_ORG + sid * rpt, rpt)])


_edge_agg = pl.kernel(
    _edge_agg_body,
    out_type=(
        jax.ShapeDtypeStruct((NC * N_ORG, D_LAB), jnp.float32),
        jax.ShapeDtypeStruct((NW * N_ORG * CSTRIDE,), jnp.float32),
    ),
    mesh=plsc.VectorSubcoreMesh(core_axis_name="c", subcore_axis_name="s",
                                num_cores=NC, num_subcores=NS),
    compiler_params=pltpu.CompilerParams(needs_layout_passes=False),
    scratch_types=[
        pltpu.VMEM((NCH, ECHUNK), jnp.int32),
        pltpu.VMEM((NCH, ECHUNK), jnp.int32),
        pltpu.VMEM((2, ECHUNK, D_LAB), jnp.float32),
        pltpu.VMEM((HWORDS,), jnp.float32),
        pltpu.VMEM_SHARED((N_ORG, D_LAB), jnp.float32),
        pltpu.SemaphoreType.DMA,
        pltpu.SemaphoreType.DMA,
        pltpu.SemaphoreType.DMA,
        pltpu.SemaphoreType.DMA,
    ],
)


def _final_body(chist_ref, part_ref, z_ref, worgt_ref, borg_ref, out_ref,
                cacc_ref):
    i = pl.program_id(0)

    @pl.when(i == 0)
    def _():
        cacc_ref[...] = jnp.zeros_like(cacc_ref)

    cacc_ref[...] += chist_ref[0]

    @pl.when(i == NW - 1)
    def _():
        agg = part_ref[0] + part_ref[1]
        out_ref[...] = (
            jnp.dot(agg, worgt_ref[...], preferred_element_type=jnp.float32, precision=lax.Precision.HIGHEST)
            + jnp.dot(cacc_ref[...], z_ref[...][:N_REL],
                      preferred_element_type=jnp.float32, precision=lax.Precision.HIGHEST)
            + borg_ref[...])


_final = pl.pallas_call(
    _final_body,
    grid=(NW,),
    in_specs=[
        pl.BlockSpec((1, N_ORG, CSTRIDE), lambda i: (i, 0, 0)),
        pl.BlockSpec((NC, N_ORG, D_LAB), lambda i: (0, 0, 0)),
        pl.BlockSpec((N_REL_PAD, D_ORG), lambda i: (0, 0)),
        pl.BlockSpec((D_LAB, D_ORG), lambda i: (0, 0)),
        pl.BlockSpec((1, D_ORG), lambda i: (0, 0)),
    ],
    out_specs=pl.BlockSpec((N_ORG, D_ORG), lambda i: (0, 0)),
    out_shape=jax.ShapeDtypeStruct((N_ORG, D_ORG), jnp.float32),
    scratch_shapes=[pltpu.VMEM((N_ORG, CSTRIDE), jnp.float32)],
)


def kernel(lab_features, lab_concept_emb, relation_emb, W_lab_w, W_lab_b, D_w,
           W_org_w, W_org_b, lab_concept_idx, edge_src, edge_dst, edge_rel):
    lab_pad = jnp.pad(lab_features, ((0, N_PAD - N), (0, 0)))
    idx_pad = jnp.pad(lab_concept_idx.astype(jnp.int32), (0, N_PAD - N))
    remb_pad = jnp.pad(relation_emb, ((0, N_REL_PAD - N_REL), (0, 0)))

    w1t = W_lab_w[:, :D_LAB].T
    w2t = W_lab_w[:, D_LAB:].T
    b2d = W_lab_b.reshape(1, D_LAB)

    src32 = edge_src.astype(jnp.int32)
    dst32 = edge_dst.astype(jnp.int32)
    blab, plab, z, keys = _dense_pre(lab_pad, w1t, b2d, lab_concept_emb, w2t,
                                     remb_pad, D_w.T, W_org_w.T,
                                     dst32.reshape(E // D_LAB, D_LAB),
                                     edge_rel.astype(jnp.int32)
                                     .reshape(E // D_LAB, D_LAB))
    h = _h_build(plab, blab, idx_pad.reshape(NW, NGCH, GCHUNK))
    parts, chist = _edge_agg(h,
                             src32.reshape(NW, NCH, ECHUNK),
                             dst32.reshape(NW, NCH, ECHUNK),
                             keys.reshape(NW, NCH, ECHUNK),
                             jnp.zeros((HWORDS,), jnp.float32),
                             jnp.zeros((N_ORG // NS, D_LAB), jnp.float32))
    part2 = parts.reshape(NC, N_ORG, D_LAB)
    chist3 = chist.reshape(NW, N_ORG, CSTRIDE)
    return _final(chist3, part2, z, W_org_w.T, W_org_b.reshape(1, D_ORG))


# speedup vs baseline: 1.2348x; 1.2348x over previous
import functools

import jax
import jax.numpy as jnp
from jax import lax
from jax.experimental import pallas as pl
from jax.experimental.pallas import tpu as pltpu
from jax.experimental.pallas import tpu_sc as plsc

N = 10000
E = 320000
D_LAB = 128
D_EMB = 768
D_ORG = 128
N_LABS = 1024
N_ORG = 2048
N_REL = 100

NC = 2
NS = 16
NW = NC * NS

N_PAD = 10240
ROWS_PER_W = N_PAD // NW
GCHUNK = 80
NGCH = ROWS_PER_W // GCHUNK
ECHUNK = 80
NCH = (E // NW) // ECHUNK
N_REL_PAD = 128

HP = 4
HORG = N_ORG // HP
CSTRIDE = N_REL
HWORDS = HORG * CSTRIDE


def _dense_pre_body(lab_ref, w1t_ref, b_ref, cemb_ref, w2t_ref, remb_ref,
                    dwt_ref, worgt_ref, dst_ref, rel_ref,
                    blab_ref, plab_ref, z_ref, key_ref):
    blab_ref[...] = jnp.dot(lab_ref[...], w1t_ref[...],
                            preferred_element_type=jnp.float32) + b_ref[...]
    plab_ref[...] = jnp.dot(cemb_ref[...], w2t_ref[...],
                            preferred_element_type=jnp.float32)
    prel = jnp.dot(remb_ref[...], dwt_ref[...],
                   preferred_element_type=jnp.float32, precision=lax.Precision.HIGHEST)
    z_ref[...] = jnp.dot(prel, worgt_ref[...],
                         preferred_element_type=jnp.float32, precision=lax.Precision.HIGHEST)
    key_ref[...] = dst_ref[...] * CSTRIDE + rel_ref[...]


_dense_pre = pl.pallas_call(
    _dense_pre_body,
    out_shape=(
        jax.ShapeDtypeStruct((N_PAD, D_LAB), jnp.float32),
        jax.ShapeDtypeStruct((N_LABS, D_LAB), jnp.float32),
        jax.ShapeDtypeStruct((N_REL_PAD, D_LAB), jnp.float32),
        jax.ShapeDtypeStruct((E // D_LAB, D_LAB), jnp.int32),
    ),
)


def _h_build_body(plab_hbm, blab_hbm, idx3d_hbm, h_hbm,
                  idx_v, g_v, b_v, semg, semb):
    cid = lax.axis_index("c")
    sid = lax.axis_index("s")
    wid = sid * NC + cid
    base = wid * ROWS_PER_W
    pltpu.sync_copy(idx3d_hbm.at[wid], idx_v)
    for ci in range(NGCH):
        pltpu.async_copy(plab_hbm.at[idx_v.at[ci]],
                         g_v.at[pl.ds(ci * GCHUNK, GCHUNK)], semg)
    pltpu.async_copy(blab_hbm.at[pl.ds(base, ROWS_PER_W)], b_v, semb)
    for ci in range(NGCH):
        pltpu.make_async_copy(plab_hbm.at[idx_v.at[ci]],
                              g_v.at[pl.ds(ci * GCHUNK, GCHUNK)], semg).wait()
    pltpu.make_async_copy(blab_hbm.at[pl.ds(base, ROWS_PER_W)], b_v,
                          semb).wait()

    def row_body(i, _):
        for j in range(D_LAB // 16):
            s = pl.ds(j * 16, 16)
            g_v[i, s] = jnp.maximum(g_v[i, s] + b_v[i, s], 0.0)
        return 0

    lax.fori_loop(0, ROWS_PER_W, row_body, 0)
    pltpu.sync_copy(g_v, h_hbm.at[pl.ds(base, ROWS_PER_W)])


_h_build = pl.kernel(
    _h_build_body,
    out_type=jax.ShapeDtypeStruct((N_PAD, D_LAB), jnp.float32),
    mesh=plsc.VectorSubcoreMesh(core_axis_name="c", subcore_axis_name="s",
                                num_cores=NC, num_subcores=NS),
    scratch_types=[
        pltpu.VMEM((NGCH, GCHUNK), jnp.int32),
        pltpu.VMEM((ROWS_PER_W, D_LAB), jnp.float32),
        pltpu.VMEM((ROWS_PER_W, D_LAB), jnp.float32),
        pltpu.SemaphoreType.DMA,
        pltpu.SemaphoreType.DMA,
    ],
)


def _edge_agg_body(h_hbm, src2d_hbm, dst2d_hbm, key_hbm, zeros_hbm, z2d_hbm,
                   out_hbm, outc_hbm, srcall_v, dstall_v, rowsa_v, hist_v,
                   agg_sh, semg0, semg1, sems0, sems1):
    sems_g = (semg0, semg1)
    sems_s = (sems0, sems1)
    cid = lax.axis_index("c")
    sid = lax.axis_index("s")
    wid = sid * NC + cid

    base_r = sid * (N_ORG // NS)
    pltpu.sync_copy(z2d_hbm, agg_sh.at[pl.ds(base_r, N_ORG // NS)])

    pltpu.sync_copy(src2d_hbm.at[wid], srcall_v)
    pltpu.sync_copy(dst2d_hbm.at[wid], dstall_v)
    plsc.subcore_barrier()

    nch = NCH

    def fire_gather(t, slot):
        pltpu.async_copy(h_hbm.at[srcall_v.at[t]], rowsa_v.at[slot],
                         sems_g[slot])

    def wait_gather(t, slot):
        pltpu.make_async_copy(h_hbm.at[srcall_v.at[t]], rowsa_v.at[slot],
                              sems_g[slot]).wait()

    def fire_scatter(t, slot):
        pltpu.async_copy(rowsa_v.at[slot], agg_sh.at[dstall_v.at[t]],
                         sems_s[slot], add=True)

    def wait_scatter(t, slot):
        pltpu.make_async_copy(rowsa_v.at[slot], agg_sh.at[dstall_v.at[t]],
                              sems_s[slot]).wait()

    fire_gather(0, 0)

    def pair(p, _):
        for half in range(2):
            t = 2 * p + half
            nslot = (half + 1) % 2

            @pl.when((t + 1 < nch) & (t >= 1))
            def _():
                wait_scatter(t - 1, nslot)

            @pl.when(t + 1 < nch)
            def _():
                fire_gather(t + 1, nslot)

            @pl.when(t < nch)
            def _():
                wait_gather(t, half)
                fire_scatter(t, half)
        return 0

    lax.fori_loop(0, (nch + 1) // 2, pair, 0)
    wait_scatter(nch - 2, (nch - 2) % 2)
    wait_scatter(nch - 1, (nch - 1) % 2)

    keyall_v = srcall_v
    pltpu.sync_copy(key_hbm.at[wid], keyall_v)

    ones16 = jnp.ones((16,), jnp.float32)

    for p in range(HP):
        pltpu.sync_copy(zeros_hbm, hist_v)
        klo = p * HORG * CSTRIDE

        def hbody(i, _):
            for g in range(ECHUNK // 16):
                k = keyall_v[i, pl.ds(g * 16, 16)]
                idx = k - klo
                m = (k >= klo) & (k < klo + HWORDS)
                plsc.addupdate_scatter(hist_v, [idx], ones16, mask=m)
            return 0

        lax.fori_loop(0, NCH, hbody, 0)
        pltpu.sync_copy(
            hist_v, outc_hbm.at[pl.ds(wid * HP * HWORDS + p * HWORDS, HWORDS)])

    plsc.subcore_barrier()

    rpt = N_ORG // NS
    pltpu.sync_copy(agg_sh.at[pl.ds(sid * rpt, rpt)],
                    out_hbm.at[pl.ds(cid * N_ORG + sid * rpt, rpt)])


_edge_agg = pl.kernel(
    _edge_agg_body,
    out_type=(
        jax.ShapeDtypeStruct((NC * N_ORG, D_LAB), jnp.float32),
        jax.ShapeDtypeStruct((NW * N_ORG * CSTRIDE,), jnp.float32),
    ),
    mesh=plsc.VectorSubcoreMesh(core_axis_name="c", subcore_axis_name="s",
                                num_cores=NC, num_subcores=NS),
    compiler_params=pltpu.CompilerParams(needs_layout_passes=False),
    scratch_types=[
        pltpu.VMEM((NCH, ECHUNK), jnp.int32),
        pltpu.VMEM((NCH, ECHUNK), jnp.int32),
        pltpu.VMEM((2, ECHUNK, D_LAB), jnp.float32),
        pltpu.VMEM((HWORDS,), jnp.float32),
        pltpu.VMEM_SHARED((N_ORG, D_LAB), jnp.float32),
        pltpu.SemaphoreType.DMA,
        pltpu.SemaphoreType.DMA,
        pltpu.SemaphoreType.DMA,
        pltpu.SemaphoreType.DMA,
    ],
)


CROWS = N_ORG * CSTRIDE // D_LAB


def _c_reduce_body(chist_ref, out_ref, cacc_ref):
    i = pl.program_id(0)

    @pl.when(i == 0)
    def _():
        cacc_ref[...] = jnp.zeros_like(cacc_ref)

    cacc_ref[...] += chist_ref[0]

    @pl.when(i == NW - 1)
    def _():
        out_ref[...] = cacc_ref[...]


_c_reduce = pl.pallas_call(
    _c_reduce_body,
    grid=(NW,),
    in_specs=[pl.BlockSpec((1, CROWS, D_LAB), lambda i: (i, 0, 0))],
    out_specs=pl.BlockSpec((CROWS, D_LAB), lambda i: (0, 0)),
    out_shape=jax.ShapeDtypeStruct((CROWS, D_LAB), jnp.float32),
    scratch_shapes=[pltpu.VMEM((CROWS, D_LAB), jnp.float32)],
)


def _final_body(c_ref, part_ref, z_ref, worgt_ref, borg_ref, out_ref):
    agg = part_ref[0] + part_ref[1]
    out_ref[...] = (
        jnp.dot(agg, worgt_ref[...], preferred_element_type=jnp.float32, precision=lax.Precision.HIGHEST)
        + jnp.dot(c_ref[...], z_ref[...][:N_REL],
                  preferred_element_type=jnp.float32, precision=lax.Precision.HIGHEST)
        + borg_ref[...])


_final = pl.pallas_call(
    _final_body,
    out_shape=jax.ShapeDtypeStruct((N_ORG, D_ORG), jnp.float32),
)


def kernel(lab_features, lab_concept_emb, relation_emb, W_lab_w, W_lab_b, D_w,
           W_org_w, W_org_b, lab_concept_idx, edge_src, edge_dst, edge_rel):
    lab_pad = jnp.pad(lab_features, ((0, N_PAD - N), (0, 0)))
    idx_pad = jnp.pad(lab_concept_idx.astype(jnp.int32), (0, N_PAD - N))
    remb_pad = jnp.pad(relation_emb, ((0, N_REL_PAD - N_REL), (0, 0)))

    w1t = W_lab_w[:, :D_LAB].T
    w2t = W_lab_w[:, D_LAB:].T
    b2d = W_lab_b.reshape(1, D_LAB)

    src32 = edge_src.astype(jnp.int32)
    dst32 = edge_dst.astype(jnp.int32)
    blab, plab, z, keys = _dense_pre(lab_pad, w1t, b2d, lab_concept_emb, w2t,
                                     remb_pad, D_w.T, W_org_w.T,
                                     dst32.reshape(E // D_LAB, D_LAB),
                                     edge_rel.astype(jnp.int32)
                                     .reshape(E // D_LAB, D_LAB))
    h = _h_build(plab, blab, idx_pad.reshape(NW, NGCH, GCHUNK))
    parts, chist = _edge_agg(h,
                             src32.reshape(NW, NCH, ECHUNK),
                             dst32.reshape(NW, NCH, ECHUNK),
                             keys.reshape(NW, NCH, ECHUNK),
                             jnp.zeros((HWORDS,), jnp.float32),
                             jnp.zeros((N_ORG // NS, D_LAB), jnp.float32))
    part2 = parts.reshape(NC, N_ORG, D_LAB)
    cred = _c_reduce(chist.reshape(NW, CROWS, D_LAB))
    c2d = cred.reshape(N_ORG, CSTRIDE)
    return _final(c2d, part2, z, W_org_w.T, W_org_b.reshape(1, D_ORG))
